# initial kernel scaffold (unmeasured)
import jax
import jax.numpy as jnp
from jax import lax
from jax.experimental import pallas as pl
from jax.experimental.pallas import tpu as pltpu

N_DEV = 16


def kernel(x, w_mat):
    m_per, k = x.shape
    _, n_per = w_mat.shape
    half = m_per // 2

    def body(x_ref, w_ref, out_ref, gather, stage_top, stage_bot,
             send_cw, recv_cw, send_ccw, recv_ccw, local_sems):
        me = lax.axis_index("i")
        left = lax.rem(me - 1 + N_DEV, N_DEV)
        right = lax.rem(me + 1, N_DEV)

        barrier = pltpu.get_barrier_semaphore()
        for nbr in (left, right):
            pl.semaphore_signal(
                barrier, inc=1,
                device_id=(nbr,), device_id_type=pl.DeviceIdType.MESH,
            )
        pl.semaphore_wait(barrier, 2)

        cpx = pltpu.make_async_copy(x_ref, gather.at[me], local_sems.at[0])
        cpx.start()
        cpx.wait()

        def compute(o_top, o_bot):
            ctop = pltpu.make_async_copy(
                gather.at[o_top, pl.ds(0, half), :], stage_top,
                local_sems.at[1])
            cbot = pltpu.make_async_copy(
                gather.at[o_bot, pl.ds(half, half), :], stage_bot,
                local_sems.at[2])
            ctop.start()
            cbot.start()
            ctop.wait()
            out_ref[pl.ds(o_top * m_per, half), :] = jnp.maximum(
                jnp.dot(stage_top[:, :], w_ref[:, :],
                        preferred_element_type=jnp.float32),
                0.0)
            cbot.wait()
            out_ref[pl.ds(o_bot * m_per + half, half), :] = jnp.maximum(
                jnp.dot(stage_bot[:, :], w_ref[:, :],
                        preferred_element_type=jnp.float32),
                0.0)

        for h in range(N_DEV - 1):
            o_cw = lax.rem(me - h + N_DEV, N_DEV)
            o_ccw = lax.rem(me + h, N_DEV)
            scw = pltpu.make_async_remote_copy(
                src_ref=gather.at[o_cw, pl.ds(0, half), :],
                dst_ref=gather.at[o_cw, pl.ds(0, half), :],
                send_sem=send_cw.at[h], recv_sem=recv_cw.at[h],
                device_id=(right,), device_id_type=pl.DeviceIdType.MESH,
            )
            sccw = pltpu.make_async_remote_copy(
                src_ref=gather.at[o_ccw, pl.ds(half, half), :],
                dst_ref=gather.at[o_ccw, pl.ds(half, half), :],
                send_sem=send_ccw.at[h], recv_sem=recv_ccw.at[h],
                device_id=(left,), device_id_type=pl.DeviceIdType.MESH,
            )
            scw.start()
            sccw.start()

            compute(o_cw, o_ccw)

            r_cw = lax.rem(me - h - 1 + N_DEV, N_DEV)
            r_ccw = lax.rem(me + h + 1, N_DEV)
            rcw = pltpu.make_async_remote_copy(
                src_ref=gather.at[r_cw, pl.ds(0, half), :],
                dst_ref=gather.at[r_cw, pl.ds(0, half), :],
                send_sem=send_cw.at[h], recv_sem=recv_cw.at[h],
                device_id=(left,), device_id_type=pl.DeviceIdType.MESH,
            )
            rccw = pltpu.make_async_remote_copy(
                src_ref=gather.at[r_ccw, pl.ds(half, half), :],
                dst_ref=gather.at[r_ccw, pl.ds(half, half), :],
                send_sem=send_ccw.at[h], recv_sem=recv_ccw.at[h],
                device_id=(right,), device_id_type=pl.DeviceIdType.MESH,
            )
            rcw.wait_recv()
            rccw.wait_recv()
            scw.wait_send()
            sccw.wait_send()

        compute(lax.rem(me + 1, N_DEV), lax.rem(me - 1 + N_DEV, N_DEV))

    return pl.pallas_call(
        body,
        out_shape=jax.ShapeDtypeStruct((N_DEV * m_per, n_per), jnp.float32),
        in_specs=[
            pl.BlockSpec(memory_space=pl.ANY),
            pl.BlockSpec(memory_space=pltpu.MemorySpace.VMEM),
        ],
        out_specs=pl.BlockSpec(memory_space=pltpu.MemorySpace.VMEM),
        scratch_shapes=[
            pltpu.MemorySpace.HBM((N_DEV, m_per, k), jnp.float32),
            pltpu.MemorySpace.VMEM((half, k), jnp.float32),
            pltpu.MemorySpace.VMEM((half, k), jnp.float32),
            pltpu.SemaphoreType.DMA((N_DEV - 1,)),
            pltpu.SemaphoreType.DMA((N_DEV - 1,)),
            pltpu.SemaphoreType.DMA((N_DEV - 1,)),
            pltpu.SemaphoreType.DMA((N_DEV - 1,)),
            pltpu.SemaphoreType.DMA((3,)),
        ],
        compiler_params=pltpu.CompilerParams(
            collective_id=0,
            vmem_limit_bytes=100 * 1024 * 1024,
        ),
    )(x, w_mat)


# baseline (device time: 1945532 ns/iter reference)
import jax
import jax.numpy as jnp
from jax import lax
from jax.experimental import pallas as pl
from jax.experimental.pallas import tpu as pltpu

N_DEV = 16


def kernel(x, w_mat):
    m_per, k = x.shape
    _, n_per = w_mat.shape
    half = m_per // 2

    def body(x_ref, w_ref, out_ref, gather, stage_top, stage_bot,
             send_cw, recv_cw, send_ccw, recv_ccw, local_sems):
        me = lax.axis_index("i")
        left = lax.rem(me - 1 + N_DEV, N_DEV)
        right = lax.rem(me + 1, N_DEV)

        barrier = pltpu.get_barrier_semaphore()
        for nbr in (left, right):
            pl.semaphore_signal(
                barrier, inc=1,
                device_id=(nbr,), device_id_type=pl.DeviceIdType.MESH,
            )
        pl.semaphore_wait(barrier, 2)

        cpx = pltpu.make_async_copy(x_ref, gather.at[me], local_sems.at[0])
        cpx.start()
        cpx.wait()

        def compute(o_top, o_bot):
            ctop = pltpu.make_async_copy(
                gather.at[o_top, pl.ds(0, half), :], stage_top,
                local_sems.at[1])
            cbot = pltpu.make_async_copy(
                gather.at[o_bot, pl.ds(half, half), :], stage_bot,
                local_sems.at[2])
            ctop.start()
            cbot.start()
            ctop.wait()
            out_ref[pl.ds(o_top * m_per, half), :] = jnp.maximum(
                jnp.dot(stage_top[:, :], w_ref[:, :],
                        preferred_element_type=jnp.float32),
                0.0)
            cbot.wait()
            out_ref[pl.ds(o_bot * m_per + half, half), :] = jnp.maximum(
                jnp.dot(stage_bot[:, :], w_ref[:, :],
                        preferred_element_type=jnp.float32),
                0.0)

        for h in range(N_DEV - 1):
            o_cw = lax.rem(me - h + N_DEV, N_DEV)
            o_ccw = lax.rem(me + h, N_DEV)
            scw = pltpu.make_async_remote_copy(
                src_ref=gather.at[o_cw, pl.ds(0, half), :],
                dst_ref=gather.at[o_cw, pl.ds(0, half), :],
                send_sem=send_cw.at[h], recv_sem=recv_cw.at[h],
                device_id=(right,), device_id_type=pl.DeviceIdType.MESH,
            )
            sccw = pltpu.make_async_remote_copy(
                src_ref=gather.at[o_ccw, pl.ds(half, half), :],
                dst_ref=gather.at[o_ccw, pl.ds(half, half), :],
                send_sem=send_ccw.at[h], recv_sem=recv_ccw.at[h],
                device_id=(left,), device_id_type=pl.DeviceIdType.MESH,
            )
            scw.start()
            sccw.start()

            compute(o_cw, o_ccw)

            r_cw = lax.rem(me - h - 1 + N_DEV, N_DEV)
            r_ccw = lax.rem(me + h + 1, N_DEV)
            rcw = pltpu.make_async_remote_copy(
                src_ref=gather.at[r_cw, pl.ds(0, half), :],
                dst_ref=gather.at[r_cw, pl.ds(0, half), :],
                send_sem=send_cw.at[h], recv_sem=recv_cw.at[h],
                device_id=(left,), device_id_type=pl.DeviceIdType.MESH,
            )
            rccw = pltpu.make_async_remote_copy(
                src_ref=gather.at[r_ccw, pl.ds(half, half), :],
                dst_ref=gather.at[r_ccw, pl.ds(half, half), :],
                send_sem=send_ccw.at[h], recv_sem=recv_ccw.at[h],
                device_id=(right,), device_id_type=pl.DeviceIdType.MESH,
            )
            rcw.wait_recv()
            rccw.wait_recv()
            scw.wait_send()
            sccw.wait_send()

        compute(lax.rem(me + 1, N_DEV), lax.rem(me - 1 + N_DEV, N_DEV))

    out, _ = pl.pallas_call(
        body,
        out_shape=[
            jax.ShapeDtypeStruct((N_DEV * m_per, n_per), jnp.float32),
            jax.ShapeDtypeStruct((N_DEV, m_per, k), jnp.float32),
        ],
        in_specs=[
            pl.BlockSpec(memory_space=pl.ANY),
            pl.BlockSpec(memory_space=pltpu.MemorySpace.VMEM),
        ],
        out_specs=[
            pl.BlockSpec(memory_space=pltpu.MemorySpace.VMEM),
            pl.BlockSpec(memory_space=pl.ANY),
        ],
        scratch_shapes=[
            pltpu.MemorySpace.VMEM((half, k), jnp.float32),
            pltpu.MemorySpace.VMEM((half, k), jnp.float32),
            pltpu.SemaphoreType.DMA((N_DEV - 1,)),
            pltpu.SemaphoreType.DMA((N_DEV - 1,)),
            pltpu.SemaphoreType.DMA((N_DEV - 1,)),
            pltpu.SemaphoreType.DMA((N_DEV - 1,)),
            pltpu.SemaphoreType.DMA((3,)),
        ],
        compiler_params=pltpu.CompilerParams(
            collective_id=0,
            vmem_limit_bytes=100 * 1024 * 1024,
        ),
    )(x, w_mat)
    return out
